# baseline (speedup 1.0000x reference)
import functools

import jax
import jax.numpy as jnp
import numpy as np
from jax import lax
from jax.experimental import pallas as pl
from jax.experimental.pallas import tpu as pltpu
from jax.experimental.pallas import tpu_sc as plsc

_VOCAB_SIZES = [100000] * 26
_EMBED = 32
_B, _T, _H = 1024, 20, 26

_TOTAL = _B * _T * _H
_NW = 32
_ROW = 128
_ROWS_PER_W = _TOTAL // (_NW * _ROW)
_K = 13
_STEPS = _ROWS_PER_W // _K

_head_offsets = np.concatenate(
    [[0], np.cumsum(np.asarray(_VOCAB_SIZES[:-1], dtype=np.int64))]
).astype(np.int32)
_j = np.arange(13)[:, None]
_l = np.arange(_ROW)[None, :]
_OFFTAB = jnp.asarray(_head_offsets[(_j * _ROW + _l) % _H], dtype=jnp.int32)

_mesh = plsc.VectorSubcoreMesh(core_axis_name="c", subcore_axis_name="s")


@functools.partial(
    pl.kernel,
    mesh=_mesh,
    out_type=jax.ShapeDtypeStruct((_TOTAL, _EMBED), jnp.float32),
    scratch_types=[
        pltpu.VMEM((_ROWS_PER_W, _ROW), jnp.int32),
        pltpu.VMEM((13, _ROW), jnp.int32),
        pltpu.VMEM((_K * _ROW, _EMBED), jnp.float32),
        pltpu.SemaphoreType.DMA,
    ],
    compiler_params=pltpu.CompilerParams(use_tc_tiling_on_sc=False),
)
def _sc_gather(table_hbm, idx_hbm, off_hbm, out_hbm, idx_v, off_v, rows_v, sem):
    wid = lax.axis_index("s") * 2 + lax.axis_index("c")
    row0 = wid * _ROWS_PER_W * _ROW

    pltpu.sync_copy(idx_hbm.at[wid], idx_v)
    pltpu.sync_copy(off_hbm, off_v)

    def add_offsets(j, carry):
        jm = lax.rem(j, 13)
        for v in range(_ROW // 16):
            sl = pl.ds(v * 16, 16)
            idx_v[j, sl] = idx_v[j, sl] + off_v[jm, sl]
        return carry

    lax.fori_loop(0, _ROWS_PER_W, add_offsets, 0)

    def step(s, carry):
        copies = []
        for k in range(_K):
            j = s * _K + k
            copies.append(
                pltpu.async_copy(
                    table_hbm.at[idx_v.at[j]],
                    rows_v.at[pl.ds(k * _ROW, _ROW)],
                    sem,
                )
            )
        for c in copies:
            c.wait()
        pltpu.sync_copy(
            rows_v, out_hbm.at[pl.ds(row0 + s * (_K * _ROW), _K * _ROW)]
        )
        return carry

    lax.fori_loop(0, _STEPS, step, 0)


def kernel(indices, table):
    idx = indices.reshape(_NW, _ROWS_PER_W, _ROW).astype(jnp.int32)
    out = _sc_gather(table, idx, _OFFTAB)
    return out.reshape(_B, _T, _H, _EMBED)
